# baseline (speedup 1.0000x reference)
import numpy as np

import jax
import jax.numpy as jnp
from jax import lax
from jax.experimental import pallas as pl
from jax.experimental.pallas import tpu as pltpu
from jax.experimental.pallas import tpu_sc as plsc

_VOCAB = 1000000
_HIDDEN = 16
_BATCH = 4096
_SEQ = 200
_B = _BATCH * _SEQ
_NC = 2
_NS = 16
_NW = _NC * _NS

_TTC = 2048
_TTR = _TTC * _HIDDEN // 128
_TTG = (_VOCAB + _TTC - 1) // _TTC

_GCH = 1024
_NGCH = _B // _GCH
_GPW = _NGCH // _NW

_OCH = 16
_OTG = _B // (128 * _OCH)

_M_NP = np.zeros((8, 16, 128), np.float32)
for _t in range(8):
    for _h in range(16):
        _M_NP[_t, _h, 16 * _t + _h] = 1.0
_M2_NP = np.zeros((8, 16, 128), np.float32)
for _bb in range(8):
    for _a in range(16):
        _M2_NP[_bb, _a, 8 * _a + _bb] = 1.0


def _table_t_body(e_ref, m_ref, out_ref):
    x = e_ref[...]
    xt3 = x.T.reshape(_TTR, 8, 16)
    acc = None
    for t in range(8):
        zt = xt3[:, t, :].reshape(_TTR, 16)
        p = jnp.dot(zt, m_ref[t], preferred_element_type=jnp.float32)
        acc = p if acc is None else acc + p
    out_ref[...] = acc


_table_t = pl.pallas_call(
    _table_t_body,
    grid=(_TTG,),
    in_specs=[
        pl.BlockSpec((_HIDDEN, _TTC), lambda i: (0, i)),
        pl.BlockSpec((8, 16, 128), lambda i: (0, 0, 0)),
    ],
    out_specs=pl.BlockSpec((_TTR, 128), lambda i: (i, 0)),
    out_shape=jax.ShapeDtypeStruct((_VOCAB * _HIDDEN // 128, 128), jnp.float32),
)


def _out_t_body(g_ref, m_ref, out_ref):
    for c in range(_OCH):
        x = g_ref[pl.ds(16 * c, 16), :]
        xt3 = x.T.reshape(8, 16, 16)
        acc = None
        for bb in range(8):
            p = jnp.dot(xt3[bb], m_ref[bb], preferred_element_type=jnp.float32)
            acc = p if acc is None else acc + p
        out_ref[:, pl.ds(8 * c, 8), :] = acc.reshape(2, 8, 128)


_out_t = pl.pallas_call(
    _out_t_body,
    grid=(_OTG,),
    in_specs=[
        pl.BlockSpec((16 * _OCH, 128), lambda i: (i, 0)),
        pl.BlockSpec((8, 16, 128), lambda i: (0, 0, 0)),
    ],
    out_specs=pl.BlockSpec((2, 8 * _OCH, 128), lambda i: (0, i, 0)),
    out_shape=jax.ShapeDtypeStruct((2, _B // 128 * 8, 128), jnp.float32),
)


def _make_gather_sc():
    import functools

    mesh = plsc.VectorSubcoreMesh(core_axis_name="c", subcore_axis_name="s")

    @functools.partial(
        pl.kernel,
        mesh=mesh,
        out_type=jax.ShapeDtypeStruct((_NGCH, _GCH, _HIDDEN), jnp.float32),
        scratch_types=[
            pltpu.VMEM((_GCH,), jnp.int32),
            pltpu.VMEM((_GCH, _HIDDEN), jnp.float32),
            pltpu.SemaphoreType.DMA,
        ],
        compiler_params=pltpu.CompilerParams(use_tc_tiling_on_sc=False),
    )
    def gather_kernel(table_hbm, idx_hbm, out_hbm, idx_v, rows_v, sem):
        wid = lax.axis_index("s") * _NC + lax.axis_index("c")
        chunk_base = wid * _GPW

        def chunk_body(i, carry):
            j = chunk_base + i
            pltpu.sync_copy(idx_hbm.at[pl.ds(j * _GCH, _GCH)], idx_v)
            pltpu.async_copy(table_hbm.at[idx_v], rows_v, sem).wait()
            pltpu.sync_copy(rows_v, out_hbm.at[j])
            return carry

        lax.fori_loop(0, _GPW, chunk_body, 0)

    return gather_kernel


_gather_sc = _make_gather_sc()


def kernel(inputs, embeddings):
    m1 = jnp.asarray(_M_NP)
    m2 = jnp.asarray(_M2_NP)
    tt = _table_t(embeddings.T, m1)
    table = tt.reshape(_VOCAB, _HIDDEN)
    g = _gather_sc(table, inputs.reshape(_B))
    x = _out_t(g.reshape(_B * _HIDDEN // 128, 128), m2)
    out2d = (
        x.reshape(2, _B // 128, 8, 128)
        .transpose(0, 2, 1, 3)
        .reshape(_HIDDEN, _B)
        .T
    )
    return out2d.reshape(_BATCH, _SEQ, _HIDDEN)


# speedup vs baseline: 1.1156x; 1.1156x over previous
import numpy as np

import jax
import jax.numpy as jnp
from jax import lax
from jax.experimental import pallas as pl
from jax.experimental.pallas import tpu as pltpu
from jax.experimental.pallas import tpu_sc as plsc

_VOCAB = 1000000
_HIDDEN = 16
_BATCH = 4096
_SEQ = 200
_B = _BATCH * _SEQ
_NC = 2
_NS = 16
_NW = _NC * _NS

_TTC = 8192
_TTR = _TTC * _HIDDEN // 128
_TTG = (_VOCAB + _TTC - 1) // _TTC

_GCH = 1024
_NGCH = _B // _GCH
_GPW = _NGCH // _NW

_OCH = 16
_OTG = _B // (128 * _OCH)

_M_NP = np.zeros((8, 16, 128), np.float32)
for _t in range(8):
    for _h in range(16):
        _M_NP[_t, _h, 16 * _t + _h] = 1.0
_M2_NP = np.zeros((8, 16, 128), np.float32)
for _bb in range(8):
    for _a in range(16):
        _M2_NP[_bb, _a, 8 * _a + _bb] = 1.0


def _table_t_body(e_ref, m_ref, out_ref):
    x = e_ref[...]
    xt3 = x.T.reshape(_TTR, 8, 16)
    out_ref[...] = jnp.concatenate(
        [xt3[:, t, :].reshape(_TTR, 16) for t in range(8)], axis=1
    )


_table_t = pl.pallas_call(
    _table_t_body,
    grid=(_TTG,),
    in_specs=[
        pl.BlockSpec((_HIDDEN, _TTC), lambda i: (0, i)),
        pl.BlockSpec((8, 16, 128), lambda i: (0, 0, 0)),
    ],
    out_specs=pl.BlockSpec((_TTR, 128), lambda i: (i, 0)),
    out_shape=jax.ShapeDtypeStruct((_VOCAB * _HIDDEN // 128, 128), jnp.float32),
)


def _out_t_body(g_ref, m_ref, out_ref):
    for c in range(_OCH):
        x = g_ref[pl.ds(16 * c, 16), :]
        xt3 = x.T.reshape(8, 16, 16)
        acc = None
        for bb in range(8):
            p = jnp.dot(xt3[bb], m_ref[bb], preferred_element_type=jnp.float32)
            acc = p if acc is None else acc + p
        out_ref[:, pl.ds(8 * c, 8), :] = acc.reshape(2, 8, 128)


_out_t = pl.pallas_call(
    _out_t_body,
    grid=(_OTG,),
    in_specs=[
        pl.BlockSpec((16 * _OCH, 128), lambda i: (i, 0)),
        pl.BlockSpec((8, 16, 128), lambda i: (0, 0, 0)),
    ],
    out_specs=pl.BlockSpec((2, 8 * _OCH, 128), lambda i: (0, i, 0)),
    out_shape=jax.ShapeDtypeStruct((2, _B // 128 * 8, 128), jnp.float32),
)


def _make_gather_sc():
    import functools

    mesh = plsc.VectorSubcoreMesh(core_axis_name="c", subcore_axis_name="s")

    @functools.partial(
        pl.kernel,
        mesh=mesh,
        out_type=jax.ShapeDtypeStruct((_NGCH, _GCH, _HIDDEN), jnp.float32),
        scratch_types=[
            pltpu.VMEM((_GCH,), jnp.int32),
            pltpu.VMEM((_GCH, _HIDDEN), jnp.float32),
            pltpu.SemaphoreType.DMA,
        ],
        compiler_params=pltpu.CompilerParams(use_tc_tiling_on_sc=False),
    )
    def gather_kernel(table_hbm, idx_hbm, out_hbm, idx_v, rows_v, sem):
        wid = lax.axis_index("s") * _NC + lax.axis_index("c")
        chunk_base = wid * _GPW

        def chunk_body(i, carry):
            j = chunk_base + i
            pltpu.sync_copy(idx_hbm.at[pl.ds(j * _GCH, _GCH)], idx_v)
            pltpu.async_copy(table_hbm.at[idx_v], rows_v, sem).wait()
            pltpu.sync_copy(rows_v, out_hbm.at[j])
            return carry

        lax.fori_loop(0, _GPW, chunk_body, 0)

    return gather_kernel


_gather_sc = _make_gather_sc()


def kernel(inputs, embeddings):
    m1 = jnp.asarray(_M_NP)
    m2 = jnp.asarray(_M2_NP)
    tt = _table_t(embeddings.T, m1)
    table = tt.reshape(_VOCAB, _HIDDEN)
    g = _gather_sc(table, inputs.reshape(_B))
    x = _out_t(g.reshape(_B * _HIDDEN // 128, 128), m2)
    out2d = (
        x.reshape(2, _B // 128, 8, 128)
        .transpose(0, 2, 1, 3)
        .reshape(_HIDDEN, _B)
        .T
    )
    return out2d.reshape(_BATCH, _SEQ, _HIDDEN)
